# baseline (speedup 1.0000x reference)
import functools

import jax
import jax.numpy as jnp
import numpy as np
from jax.experimental import pallas as pl
from jax.experimental.pallas import tpu as pltpu

B = 4
S = 1024
ROOT = 32
BLK = S // ROOT
D = 1024
DA = 256
H = 16
DH = D // H
TQ = 256
NQ = S // TQ

_INV_SQRT_DA = 1.0 / np.sqrt(DA).astype(np.float32)
_INV_SQRT_DH = 1.0 / np.sqrt(DH).astype(np.float32)


def _dot(a, b):
    return jax.lax.dot_general(
        a, b, (((1,), (0,)), ((), ())), preferred_element_type=jnp.float32)


def _dot_t(a, b):
    return jax.lax.dot_general(
        a, b, (((1,), (1,)), ((), ())), preferred_element_type=jnp.float32)


def _proj_kernel(x_ref, w_ref, o_ref):
    o_ref[...] = _dot(x_ref[...], w_ref[...])


def _select_kernel(x_ref, qm_ref, p_ref, wu_ref, wk_ref, allow_ref):
    xb = x_ref[0]
    blocks = _dot(p_ref[...], xb)
    root_emb = _dot(blocks, wu_ref[...])
    k_mat = _dot(root_emb, wk_ref[...])
    logits = _dot_t(qm_ref[0], k_mat) * _INV_SQRT_DA
    m = jnp.max(logits, axis=-1, keepdims=True)
    e = jnp.exp(logits - m)
    prob = e / jnp.sum(e, axis=-1, keepdims=True)
    row_blk = jax.lax.broadcasted_iota(jnp.int32, (S, ROOT), 0) // BLK
    col_blk = jax.lax.broadcasted_iota(jnp.int32, (S, ROOT), 1)
    allowed = (prob >= 0.5) | (row_blk == col_blk)
    allow_ref[0] = allowed.astype(jnp.float32)


def _attn_kernel(q_ref, k_ref, v_ref, a_ref, e_ref, o_ref):
    mask_tok = _dot(a_ref[0], e_ref[...])
    neg = jnp.where(mask_tok > 0.5, 0.0, -1e30).astype(jnp.float32)
    kb = k_ref[0]
    vb = v_ref[0]
    qb = q_ref[0]
    for h in range(H):
        sl = slice(h * DH, (h + 1) * DH)
        scores = _dot_t(qb[:, sl], kb[:, sl]) * _INV_SQRT_DH
        scores = scores + neg
        m = jnp.max(scores, axis=-1, keepdims=True)
        p = jnp.exp(scores - m)
        s = jnp.sum(p, axis=-1, keepdims=True)
        o_ref[0, :, sl] = _dot(p, vb[:, sl]) / s


def _outproj_kernel(a_ref, w_ref, x_ref, o_ref):
    o_ref[...] = _dot(a_ref[...], w_ref[...]) + x_ref[...]


def kernel(x, W_upd, W_key, W_query, Wq, Wk, Wv, Wo):
    f32 = jnp.float32
    x2d = x.reshape(B * S, D)

    w_all = jnp.concatenate(
        [Wq.T, Wk.T, Wv.T, W_query.T], axis=1)
    NW = 3 * D + DA
    TM, TN = 256, 256
    proj = pl.pallas_call(
        _proj_kernel,
        grid=(B * S // TM, NW // TN),
        in_specs=[
            pl.BlockSpec((TM, D), lambda i, j: (i, 0)),
            pl.BlockSpec((D, TN), lambda i, j: (0, j)),
        ],
        out_specs=pl.BlockSpec((TM, TN), lambda i, j: (i, j)),
        out_shape=jax.ShapeDtypeStruct((B * S, NW), f32),
    )(x2d, w_all)
    q = proj[:, 0 * D:1 * D].reshape(B, S, D)
    k = proj[:, 1 * D:2 * D].reshape(B, S, D)
    v = proj[:, 2 * D:3 * D].reshape(B, S, D)
    qm = proj[:, 3 * D:3 * D + DA].reshape(B, S, DA)

    pool = (jax.lax.broadcasted_iota(jnp.int32, (ROOT, S), 1) // BLK ==
            jax.lax.broadcasted_iota(jnp.int32, (ROOT, S), 0)
            ).astype(f32) / BLK
    allow = pl.pallas_call(
        _select_kernel,
        grid=(B,),
        in_specs=[
            pl.BlockSpec((1, S, D), lambda b: (b, 0, 0)),
            pl.BlockSpec((1, S, DA), lambda b: (b, 0, 0)),
            pl.BlockSpec((ROOT, S), lambda b: (0, 0)),
            pl.BlockSpec((D, D), lambda b: (0, 0)),
            pl.BlockSpec((D, DA), lambda b: (0, 0)),
        ],
        out_specs=pl.BlockSpec((1, S, ROOT), lambda b: (b, 0, 0)),
        out_shape=jax.ShapeDtypeStruct((B, S, ROOT), f32),
    )(x, qm, pool, W_upd.T, W_key.T)

    expand = (jax.lax.broadcasted_iota(jnp.int32, (ROOT, S), 1) // BLK ==
              jax.lax.broadcasted_iota(jnp.int32, (ROOT, S), 0)
              ).astype(f32)
    attn = pl.pallas_call(
        _attn_kernel,
        grid=(B, NQ),
        in_specs=[
            pl.BlockSpec((1, TQ, D), lambda b, t: (b, t, 0)),
            pl.BlockSpec((1, S, D), lambda b, t: (b, 0, 0)),
            pl.BlockSpec((1, S, D), lambda b, t: (b, 0, 0)),
            pl.BlockSpec((1, TQ, ROOT), lambda b, t: (b, t, 0)),
            pl.BlockSpec((ROOT, S), lambda b, t: (0, 0)),
        ],
        out_specs=pl.BlockSpec((1, TQ, D), lambda b, t: (b, t, 0)),
        out_shape=jax.ShapeDtypeStruct((B, S, D), f32),
    )(q, k, v, allow, expand)

    out = pl.pallas_call(
        _outproj_kernel,
        grid=(B * S // TM, D // TN),
        in_specs=[
            pl.BlockSpec((TM, D), lambda i, j: (i, 0)),
            pl.BlockSpec((D, TN), lambda i, j: (0, j)),
            pl.BlockSpec((TM, TN), lambda i, j: (i, j)),
        ],
        out_specs=pl.BlockSpec((TM, TN), lambda i, j: (i, j)),
        out_shape=jax.ShapeDtypeStruct((B * S, D), f32),
    )(attn.reshape(B * S, D), Wo.T, x2d)
    return out.reshape(B, S, D)


# speedup vs baseline: 1.5071x; 1.5071x over previous
import jax
import jax.numpy as jnp
import numpy as np
from jax.experimental import pallas as pl
from jax.experimental.pallas import tpu as pltpu

B = 4
S = 1024
ROOT = 32
BLK = S // ROOT
D = 1024
DA = 256
H = 16
DH = D // H
TQ = 256
NQ = S // TQ
TK = 256
NK = S // TK

_INV_SQRT_DA = np.float32(1.0 / np.sqrt(DA))
_INV_SQRT_DH = np.float32(1.0 / np.sqrt(DH))


def _dot(a, b):
    return jax.lax.dot_general(
        a, b, (((1,), (0,)), ((), ())), preferred_element_type=jnp.float32)


def _dot_t(a, b):
    return jax.lax.dot_general(
        a, b, (((1,), (1,)), ((), ())), preferred_element_type=jnp.float32)


def _proj_kernel(x_ref, w_ref, o_ref):
    o_ref[...] = _dot(x_ref[...], w_ref[...])


def _select_kernel(x_ref, qm_ref, p_ref, wu_ref, wk_ref, rq_ref, ck_ref,
                   allow_ref, needed_ref):
    xb = x_ref[0]
    blocks = _dot(p_ref[...], xb)
    root_emb = _dot(blocks, wu_ref[...])
    k_mat = _dot(root_emb, wk_ref[...])
    logits = _dot_t(qm_ref[0], k_mat) * _INV_SQRT_DA
    m = jnp.max(logits, axis=-1, keepdims=True)
    e = jnp.exp(logits - m)
    prob = e / jnp.sum(e, axis=-1, keepdims=True)
    row_blk = jax.lax.broadcasted_iota(jnp.int32, (S, ROOT), 0) // BLK
    col_blk = jax.lax.broadcasted_iota(jnp.int32, (S, ROOT), 1)
    allowed = ((prob >= 0.5) | (row_blk == col_blk)).astype(jnp.float32)
    allow_ref[0] = allowed
    counts = _dot(_dot(rq_ref[...], allowed), ck_ref[...])
    needed_ref[0] = (counts > 0.0).astype(jnp.int32)


def _attn_kernel(needed_ref, q_ref, k_ref, v_ref, a_ref, e_ref, o_ref,
                 acc_ref, m_ref, l_ref):
    b = pl.program_id(0)
    qt = pl.program_id(1)
    a0 = a_ref[0]

    def _neg(e_cols):
        return jnp.where(_dot(a0, e_cols) > 0.5, 0.0, -1e30
                         ).astype(jnp.float32)

    qb = q_ref[0]
    diag = pl.ds(qt * TK, TK)
    kd = k_ref[0, diag, :]
    vd = v_ref[0, diag, :]
    neg_d = _neg(e_ref[:, diag])
    neg_off = [_neg(e_ref[:, kt * TK:(kt + 1) * TK]) for kt in range(NK)]
    for h in range(H):
        sl = slice(h * DH, (h + 1) * DH)
        qh = qb[:, sl]
        chunk = _dot_t(qh, kd[:, sl]) * _INV_SQRT_DH + neg_d
        m0 = jnp.max(chunk, axis=-1, keepdims=True)
        p = jnp.exp(chunk - m0)
        m_ref[...] = m0
        l_ref[...] = jnp.sum(p, axis=-1, keepdims=True)
        acc_ref[...] = _dot(p, vd[:, sl])
        for kt in range(NK):
            cond = (needed_ref[b, qt, kt] != 0) & (kt != qt)

            @pl.when(cond)
            def _():
                ks = pl.ds(kt * TK, TK)
                ck = _dot_t(qh, k_ref[0, ks, sl]) * _INV_SQRT_DH \
                    + neg_off[kt]
                mc = jnp.max(ck, axis=-1, keepdims=True)
                m_new = jnp.maximum(m_ref[...], mc)
                alpha = jnp.exp(m_ref[...] - m_new)
                pc = jnp.exp(ck - m_new)
                acc_ref[...] = acc_ref[...] * alpha + _dot(pc, v_ref[0, ks, sl])
                l_ref[...] = l_ref[...] * alpha \
                    + jnp.sum(pc, axis=-1, keepdims=True)
                m_ref[...] = m_new

        o_ref[0, :, sl] = acc_ref[...] / l_ref[...]


def _outproj_kernel(a_ref, w_ref, x_ref, o_ref):
    o_ref[...] = _dot(a_ref[...], w_ref[...]) + x_ref[...]


def kernel(x, W_upd, W_key, W_query, Wq, Wk, Wv, Wo):
    f32 = jnp.float32
    x2d = x.reshape(B * S, D)

    w_all = jnp.concatenate(
        [Wq.T, Wk.T, Wv.T, W_query.T], axis=1)
    NW = 3 * D + DA
    TN = 256
    proj = pl.pallas_call(
        _proj_kernel,
        grid=(NW // TN,),
        in_specs=[
            pl.BlockSpec((B * S, D), lambda j: (0, 0)),
            pl.BlockSpec((D, TN), lambda j: (0, j)),
        ],
        out_specs=pl.BlockSpec((B * S, TN), lambda j: (0, j)),
        out_shape=jax.ShapeDtypeStruct((B * S, NW), f32),
    )(x2d, w_all)
    q = proj[:, 0 * D:1 * D].reshape(B, S, D)
    k = proj[:, 1 * D:2 * D].reshape(B, S, D)
    v = proj[:, 2 * D:3 * D].reshape(B, S, D)
    qm = proj[:, 3 * D:3 * D + DA].reshape(B, S, DA)

    pool = (jax.lax.broadcasted_iota(jnp.int32, (ROOT, S), 1) // BLK ==
            jax.lax.broadcasted_iota(jnp.int32, (ROOT, S), 0)
            ).astype(f32) / BLK
    rq = (jax.lax.broadcasted_iota(jnp.int32, (NQ, S), 1) // TQ ==
          jax.lax.broadcasted_iota(jnp.int32, (NQ, S), 0)).astype(f32)
    ck = (jax.lax.broadcasted_iota(jnp.int32, (ROOT, NK), 0) // (TK // BLK) ==
          jax.lax.broadcasted_iota(jnp.int32, (ROOT, NK), 1)).astype(f32)
    allow, needed = pl.pallas_call(
        _select_kernel,
        grid=(B,),
        in_specs=[
            pl.BlockSpec((1, S, D), lambda b: (b, 0, 0)),
            pl.BlockSpec((1, S, DA), lambda b: (b, 0, 0)),
            pl.BlockSpec((ROOT, S), lambda b: (0, 0)),
            pl.BlockSpec((D, D), lambda b: (0, 0)),
            pl.BlockSpec((D, DA), lambda b: (0, 0)),
            pl.BlockSpec((NQ, S), lambda b: (0, 0)),
            pl.BlockSpec((ROOT, NK), lambda b: (0, 0)),
        ],
        out_specs=[
            pl.BlockSpec((1, S, ROOT), lambda b: (b, 0, 0)),
            pl.BlockSpec((1, NQ, NK), lambda b: (b, 0, 0)),
        ],
        out_shape=[
            jax.ShapeDtypeStruct((B, S, ROOT), f32),
            jax.ShapeDtypeStruct((B, NQ, NK), jnp.int32),
        ],
    )(x, qm, pool, W_upd.T, W_key.T, rq, ck)

    expand = (jax.lax.broadcasted_iota(jnp.int32, (ROOT, S), 1) // BLK ==
              jax.lax.broadcasted_iota(jnp.int32, (ROOT, S), 0)
              ).astype(f32)
    attn = pl.pallas_call(
        _attn_kernel,
        grid=(B, NQ),
        in_specs=[
            pl.BlockSpec(memory_space=pltpu.SMEM),
            pl.BlockSpec((1, TQ, D), lambda b, t: (b, t, 0)),
            pl.BlockSpec((1, S, D), lambda b, t: (b, 0, 0)),
            pl.BlockSpec((1, S, D), lambda b, t: (b, 0, 0)),
            pl.BlockSpec((1, TQ, ROOT), lambda b, t: (b, t, 0)),
            pl.BlockSpec((ROOT, S), lambda b, t: (0, 0)),
        ],
        out_specs=pl.BlockSpec((1, TQ, D), lambda b, t: (b, t, 0)),
        out_shape=jax.ShapeDtypeStruct((B, S, D), f32),
        scratch_shapes=[
            pltpu.VMEM((TQ, DH), f32),
            pltpu.VMEM((TQ, 1), f32),
            pltpu.VMEM((TQ, 1), f32),
        ],
    )(needed, q, k, v, allow, expand)

    out = pl.pallas_call(
        _outproj_kernel,
        grid=(D // TN,),
        in_specs=[
            pl.BlockSpec((B * S, D), lambda j: (0, 0)),
            pl.BlockSpec((D, TN), lambda j: (0, j)),
            pl.BlockSpec((B * S, TN), lambda j: (0, j)),
        ],
        out_specs=pl.BlockSpec((B * S, TN), lambda j: (0, j)),
        out_shape=jax.ShapeDtypeStruct((B * S, D), f32),
    )(attn.reshape(B * S, D), Wo.T, x2d)
    return out.reshape(B, S, D)
